# baseline (speedup 1.0000x reference)
import functools

import jax
import jax.numpy as jnp
from jax import lax
from jax.experimental import pallas as pl
from jax.experimental.pallas import tpu as pltpu
from jax.experimental.pallas import tpu_sc as plsc

M, C_IN, C_OUT, F, FP, K = 2, 128, 128, 50000, 25000, 16
G = K + 1

NC, NS = 2, 16
NW = NC * NS
FP_PAD = 25088
TOT = M * FP_PAD
RPT = TOT // NW
FG = 7
GROW = FG * G + 1
NGRP = RPT // FG
NB = 112
GPC = NB // FG
CPT = RPT // NB
RB = 4

F_PAD = 50176
BF = 3584
BFP4 = 1792


def _mm_body(fea_ref, w_ref, out_ref):
    x = fea_ref[0]
    w = w_ref[...]
    out_ref[0] = lax.dot_general(
        x, w, (((0,), (1,)), ((), ())), preferred_element_type=jnp.float32)
    nj = pl.num_programs(1)

    @pl.when(pl.program_id(1) == nj - 1)
    def _():
        out_ref[0, pl.ds(F - (nj - 1) * BF, F_PAD - F), :] = jnp.zeros(
            (F_PAD - F, C_OUT), jnp.float32)


def _matmul(fea, W):
    return pl.pallas_call(
        _mm_body,
        grid=(M, F_PAD // BF),
        in_specs=[
            pl.BlockSpec((1, C_IN, BF), lambda m, j: (m, 0, j)),
            pl.BlockSpec((C_OUT, C_IN), lambda m, j: (0, 0)),
        ],
        out_specs=pl.BlockSpec((1, BF, C_OUT), lambda m, j: (m, j, 0)),
        out_shape=jax.ShapeDtypeStruct((M, F_PAD, C_OUT), jnp.float32),
    )(fea, W)


_mesh = plsc.VectorSubcoreMesh(core_axis_name="c", subcore_axis_name="s")


@functools.partial(
    pl.kernel,
    mesh=_mesh,
    out_type=(
        jax.ShapeDtypeStruct((TOT, C_OUT), jnp.float32),
        jax.ShapeDtypeStruct((NW, 2, C_OUT), jnp.float32),
    ),
    scratch_types=[
        pltpu.VMEM((NGRP, GROW), jnp.int32),
        pltpu.VMEM((RB, GROW, C_OUT), jnp.float32),
        pltpu.VMEM((2, NB, C_OUT), jnp.float32),
        pltpu.VMEM((2, C_OUT), jnp.float32),
        pltpu.SemaphoreType.DMA,
        pltpu.SemaphoreType.DMA,
    ],
)
def _gather_sum(pre_hbm, idx_hbm, out_hbm, stats_hbm,
                idx_v, gbuf, out_v, stats_v, gsem, ssem):
    wid = lax.axis_index("s") * NC + lax.axis_index("c")
    gbase = wid * NGRP
    rbase = wid * RPT

    pltpu.sync_copy(idx_hbm.at[pl.ds(gbase, NGRP)], idx_v)

    def _gather(g):
        r = lax.rem(g, RB)
        return pltpu.make_async_copy(
            pre_hbm.at[idx_v.at[g]], gbuf.at[r], gsem)

    def _store(c):
        return pltpu.make_async_copy(
            out_v.at[lax.rem(c, 2)],
            out_hbm.at[pl.ds(rbase + c * NB, NB)], ssem)

    for g0 in range(RB - 1):
        _gather(g0).start()

    zero = jnp.zeros((16,), jnp.float32)

    def g_body(g, carry):
        s_acc = carry[0:8]
        q_acc = carry[8:16]
        r = lax.rem(g, RB)
        gc = lax.rem(g, GPC)
        c = lax.div(g, GPC)

        @pl.when((gc == 0) & (g >= 2 * GPC))
        def _():
            _store(0).wait()

        _gather(g).wait()

        @pl.when(g + (RB - 1) < NGRP)
        def _():
            _gather(g + (RB - 1)).start()

        ob = lax.rem(c, 2)
        new_s = list(s_acc)
        new_q = list(q_acc)
        for f in range(FG):
            base = f * G
            i = gc * FG + f
            for c8 in range(8):
                sl = pl.ds(c8 * 16, 16)
                v = gbuf[r, base, sl]
                for k in range(1, G):
                    v = v + gbuf[r, base + k, sl]
                out_v[ob, i, sl] = v
                new_s[c8] = new_s[c8] + v
                new_q[c8] = new_q[c8] + v * v

        @pl.when(gc == GPC - 1)
        def _():
            _store(c).start()

        return tuple(new_s) + tuple(new_q)

    carry = lax.fori_loop(0, NGRP, g_body, (zero,) * 16, unroll=False)

    _store(0).wait()
    _store(0).wait()

    for c8 in range(8):
        sl = pl.ds(c8 * 16, 16)
        stats_v[0, sl] = carry[c8]
        stats_v[1, sl] = carry[8 + c8]
    pltpu.sync_copy(stats_v, stats_hbm.at[wid])


def _norm_body(s_ref, stats_ref, gamma_ref, beta_ref, out_ref, ac_ref):
    @pl.when((pl.program_id(0) == 0) & (pl.program_id(1) == 0))
    def _():
        p = jnp.sum(stats_ref[...], axis=0)
        n = float(M * FP)
        mean = p[0:1] / n
        var = p[1:2] / n - mean * mean
        a = gamma_ref[...] * lax.rsqrt(var + 1e-5)
        ac_ref[0:1] = a
        ac_ref[1:2] = beta_ref[...] - mean * a

    x = s_ref[0]
    y = jnp.maximum(x * ac_ref[0:1] + ac_ref[1:2], 0.0)
    out_ref[0] = y.T


def _normalize(s3, stats, gamma, beta):
    return pl.pallas_call(
        _norm_body,
        grid=(M, FP_PAD // BFP4),
        in_specs=[
            pl.BlockSpec((1, BFP4, C_OUT), lambda m, j: (m, j, 0)),
            pl.BlockSpec((NW, 2, C_OUT), lambda m, j: (0, 0, 0)),
            pl.BlockSpec((1, C_OUT), lambda m, j: (0, 0)),
            pl.BlockSpec((1, C_OUT), lambda m, j: (0, 0)),
        ],
        out_specs=pl.BlockSpec((1, C_OUT, BFP4), lambda m, j: (m, 0, j)),
        out_shape=jax.ShapeDtypeStruct((M, C_OUT, FP), jnp.float32),
        scratch_shapes=[pltpu.VMEM((2, C_OUT), jnp.float32)],
    )(s3, stats, gamma, beta)


def kernel(fea, ring_n, pool_idx, W, b, gamma, beta):
    del b
    pre = _matmul(fea, W).reshape(M * F_PAD, C_OUT)

    pool_b = jnp.broadcast_to(pool_idx[None, :, None], (M, FP, 1))
    idx_all = jnp.concatenate([pool_b, ring_n], axis=2)
    idx_all = jnp.pad(idx_all, ((0, 0), (0, FP_PAD - FP), (0, 0)),
                      constant_values=F)
    idx_all = idx_all + (jnp.arange(M, dtype=jnp.int32) * F_PAD)[:, None, None]
    idx_grp = idx_all.reshape(TOT // FG, FG * G)
    idx_grp = jnp.pad(idx_grp, ((0, 0), (0, GROW - FG * G)))

    s, stats = _gather_sum(pre, idx_grp)
    s3 = s.reshape(M, FP_PAD, C_OUT)
    return _normalize(s3, stats, gamma.reshape(1, C_OUT),
                      beta.reshape(1, C_OUT))
